# Initial kernel scaffold; baseline (speedup 1.0000x reference)
import jax
import jax.numpy as jnp
from jax.experimental import pallas as pl


def kernel(input_ids, W_frozen, W_trainable):
    raise NotImplementedError("write your pallas kernel here")


# speedup vs baseline: 3.6046x; 3.6046x over previous
import functools

import jax
import jax.numpy as jnp
from jax import lax
from jax.experimental import pallas as pl
from jax.experimental.pallas import tpu as pltpu
from jax.experimental.pallas import tpu_sc as plsc

FREEZE_UNTIL = 500000
EMBED_DIM = 64
NUM_CORES = 2
NUM_SUBCORES = 16
NUM_WORKERS = NUM_CORES * NUM_SUBCORES
LANES = 16
CHUNK = 128


def _lane_broadcast(vec, lane):
    idx = jnp.full((LANES, 1), lane, jnp.int32)
    dnums = lax.GatherDimensionNumbers(
        offset_dims=(), collapsed_slice_dims=(0,), start_index_map=(0,))
    return lax.gather(vec, idx, dnums, (1,),
                      mode=lax.GatherScatterMode.PROMISE_IN_BOUNDS)


def _lookup(ids_flat, w_frozen, w_trainable):
    n = ids_flat.shape[0]
    per_w = n // NUM_WORKERS
    n_chunks = per_w // CHUNK
    assert per_w * NUM_WORKERS == n and n_chunks * CHUNK == per_w

    mesh = plsc.VectorSubcoreMesh(core_axis_name="c", subcore_axis_name="s")

    @functools.partial(
        pl.kernel,
        out_type=jax.ShapeDtypeStruct((n, EMBED_DIM), jnp.float32),
        mesh=mesh,
        compiler_params=pltpu.CompilerParams(use_tc_tiling_on_sc=False),
        scratch_types=[
            pltpu.VMEM((CHUNK,), jnp.int32),
            pltpu.VMEM((CHUNK,), jnp.int32),
            pltpu.VMEM((CHUNK,), jnp.int32),
            pltpu.VMEM((CHUNK,), jnp.int32),
            pltpu.VMEM((CHUNK, EMBED_DIM), jnp.float32),
            pltpu.VMEM((CHUNK, EMBED_DIM), jnp.float32),
            pltpu.SemaphoreType.DMA,
            pltpu.SemaphoreType.DMA,
        ],
    )
    def body(ids_hbm, wf_hbm, wt_hbm, out_hbm,
             ids_v, mask_v, fidx_v, tidx_v, rows_a, rows_b, sem_a, sem_b):
        wid = lax.axis_index("s") * NUM_CORES + lax.axis_index("c")
        base = wid * per_w

        def chunk_body(k, carry):
            off = base + k * CHUNK
            pltpu.sync_copy(ids_hbm.at[pl.ds(off, CHUNK)], ids_v)
            for g in range(CHUNK // LANES):
                v = ids_v[pl.ds(g * LANES, LANES)]
                d = v - FREEZE_UNTIL
                neg = lax.shift_right_arithmetic(d, 31)
                mask_v[pl.ds(g * LANES, LANES)] = neg
                fidx_v[pl.ds(g * LANES, LANES)] = lax.bitwise_and(v, neg)
                tidx_v[pl.ds(g * LANES, LANES)] = lax.bitwise_and(
                    d, lax.bitwise_not(neg))
            cp_a = pltpu.async_copy(wf_hbm.at[fidx_v], rows_a, sem_a)
            cp_b = pltpu.async_copy(wt_hbm.at[tidx_v], rows_b, sem_b)
            cp_a.wait()
            cp_b.wait()

            def select_group(g, c2):
                m16 = mask_v[pl.ds(g * LANES, LANES)]
                for r in range(LANES):
                    m_spl = _lane_broadcast(m16, r)
                    i = g * LANES + r
                    for w in range(EMBED_DIM // LANES):
                        ai = lax.bitcast_convert_type(
                            rows_a[i, pl.ds(w * LANES, LANES)], jnp.int32)
                        bi = lax.bitcast_convert_type(
                            rows_b[i, pl.ds(w * LANES, LANES)], jnp.int32)
                        sel = lax.bitwise_xor(
                            bi, lax.bitwise_and(lax.bitwise_xor(ai, bi), m_spl))
                        rows_a[i, pl.ds(w * LANES, LANES)] = (
                            lax.bitcast_convert_type(sel, jnp.float32))
                return c2

            lax.fori_loop(0, CHUNK // LANES, select_group, 0)
            pltpu.sync_copy(rows_a, out_hbm.at[pl.ds(off, CHUNK)])
            return carry

        lax.fori_loop(0, n_chunks, chunk_body, 0)

    return body(ids_flat, w_frozen, w_trainable)


def kernel(input_ids, W_frozen, W_trainable):
    ids_flat = input_ids.reshape(-1)
    out = _lookup(ids_flat, W_frozen, W_trainable)
    return out.reshape(input_ids.shape + (EMBED_DIM,))
